# baseline (speedup 1.0000x reference)
import functools

import jax
import jax.numpy as jnp
from jax import lax
from jax.experimental import pallas as pl
from jax.experimental.pallas import tpu as pltpu
from jax.experimental.pallas import tpu_sc as plsc

N_E = 10000
N_P = 10000
E = 320000
D = 128
C = 16
N_SEEDS = 160

NC = 2
NS = 16
NW = NC * NS
CHUNK = 125
N_CHUNKS = 80
EDGES_PER_TILE = E // NW


@functools.cache
def _make_seg_sum(n_seg, with_seeds):
  mesh = plsc.VectorSubcoreMesh(core_axis_name="c", subcore_axis_name="s")
  out_type = [jax.ShapeDtypeStruct((NC, n_seg, D), jnp.float32)]
  if with_seeds:
    out_type.append(jax.ShapeDtypeStruct((N_SEEDS, D), jnp.float32))
  rows_a = 632
  rows_b = n_seg - (NS - 1) * rows_a

  @functools.partial(
      pl.kernel,
      mesh=mesh,
      out_type=tuple(out_type),
      scratch_types=[
          pltpu.VMEM((N_CHUNKS, CHUNK), jnp.int32),
          pltpu.VMEM((N_CHUNKS, CHUNK), jnp.int32),
          pltpu.VMEM((CHUNK, D), jnp.float32),
          pltpu.VMEM_SHARED((n_seg, D), jnp.float32),
          pltpu.SemaphoreType.DMA,
      ],
  )
  def seg_sum(*refs):
    if with_seeds:
      (vals_hbm, sidx_hbm, didx_hbm, zeros_hbm, seeds_hbm,
       out_hbm, seed_out_hbm, S, I, rows, acc, sem) = refs
    else:
      (vals_hbm, sidx_hbm, didx_hbm, zeros_hbm,
       out_hbm, S, I, rows, acc, sem) = refs
    cid = lax.axis_index("c")
    sid = lax.axis_index("s")
    wid = cid * NS + sid
    r0 = pl.multiple_of(sid * rows_a, 8)

    pltpu.sync_copy(sidx_hbm.at[wid, pl.ds(0, N_CHUNKS)], S)
    pltpu.sync_copy(didx_hbm.at[wid, pl.ds(0, N_CHUNKS)], I)

    @pl.when(sid < NS - 1)
    def _zero_a():
      pltpu.sync_copy(zeros_hbm.at[pl.ds(0, rows_a)],
                      acc.at[pl.ds(r0, rows_a)])

    @pl.when(sid == NS - 1)
    def _zero_b():
      pltpu.sync_copy(zeros_hbm.at[pl.ds(0, rows_b)],
                      acc.at[pl.ds(r0, rows_b)])

    plsc.subcore_barrier()

    def chunk_body(j, carry):
      pltpu.async_copy(vals_hbm.at[S.at[j]], rows, sem).wait()
      pltpu.sync_copy(rows, acc.at[I.at[j]], add=True)
      return carry

    lax.fori_loop(0, N_CHUNKS, chunk_body, 0)

    if with_seeds:
      @pl.when(wid == 0)
      def _gather_seeds():
        for off in (0, N_SEEDS // 2):
          pltpu.sync_copy(seeds_hbm.at[pl.ds(off, 80)], S.at[0, pl.ds(0, 80)])
          pltpu.async_copy(vals_hbm.at[S.at[0, pl.ds(0, 80)]],
                           rows.at[pl.ds(0, 80)], sem).wait()
          pltpu.sync_copy(rows.at[pl.ds(0, 80)],
                          seed_out_hbm.at[pl.ds(off, 80)])

    plsc.subcore_barrier()

    @pl.when(sid < NS - 1)
    def _write_a():
      pltpu.sync_copy(acc.at[pl.ds(r0, rows_a)],
                      out_hbm.at[cid, pl.ds(r0, rows_a)])

    @pl.when(sid == NS - 1)
    def _write_b():
      pltpu.sync_copy(acc.at[pl.ds(r0, rows_b)],
                      out_hbm.at[cid, pl.ds(r0, rows_b)])

  return seg_sum


_BLK = 1000


def _dense_pattern(parts, W, b):
  n = parts.shape[1]

  def body(p0_ref, p1_ref, w_ref, b_ref, o_ref):
    s = p0_ref[0] + p1_ref[0]
    y = jnp.dot(s, w_ref[...], preferred_element_type=jnp.float32,
                precision=lax.Precision.HIGHEST)
    o_ref[...] = jnp.maximum(y + b_ref[...], 0.0)

  return pl.pallas_call(
      body,
      grid=(n // _BLK,),
      in_specs=[
          pl.BlockSpec((1, _BLK, D), lambda i: (0, i, 0)),
          pl.BlockSpec((1, _BLK, D), lambda i: (1, i, 0)),
          pl.BlockSpec((D, D), lambda i: (0, 0)),
          pl.BlockSpec((1, D), lambda i: (0, 0)),
      ],
      out_specs=pl.BlockSpec((_BLK, D), lambda i: (i, 0)),
      out_shape=jax.ShapeDtypeStruct((n, D), jnp.float32),
  )(parts, parts, W, b.reshape(1, D))


def _dense_entity(parts, seed_rows, W_e, b_e, W_g):
  n = parts.shape[1]

  def body(e0_ref, e1_ref, sr_ref, we_ref, be_ref, wg_ref, o_ref):
    g = jnp.sum(sr_ref[...], axis=0, keepdims=True) * (1.0 / N_SEEDS)
    gw = jnp.dot(g, wg_ref[...], preferred_element_type=jnp.float32,
                 precision=lax.Precision.HIGHEST)
    s = e0_ref[0] + e1_ref[0]
    y = jnp.dot(s, we_ref[...], preferred_element_type=jnp.float32,
                precision=lax.Precision.HIGHEST)
    o_ref[...] = jnp.maximum(y + gw + be_ref[...], 0.0)

  return pl.pallas_call(
      body,
      grid=(n // _BLK,),
      in_specs=[
          pl.BlockSpec((1, _BLK, D), lambda i: (0, i, 0)),
          pl.BlockSpec((1, _BLK, D), lambda i: (1, i, 0)),
          pl.BlockSpec((N_SEEDS, D), lambda i: (0, 0)),
          pl.BlockSpec((D, D), lambda i: (0, 0)),
          pl.BlockSpec((1, D), lambda i: (0, 0)),
          pl.BlockSpec((D, D), lambda i: (0, 0)),
      ],
      out_specs=pl.BlockSpec((_BLK, D), lambda i: (i, 0)),
      out_shape=jax.ShapeDtypeStruct((n, D), jnp.float32),
  )(parts, parts, seed_rows, W_e, b_e.reshape(1, D), W_g)


def _dense_entity_fc(parts, seed_rows, W_e, b_e, W_g, W_fc_pad, b_fc_pad):
  n = parts.shape[1]

  def body(e0_ref, e1_ref, sr_ref, we_ref, be_ref, wg_ref, wfc_ref, bfc_ref,
           es_ref, out_ref):
    g = jnp.sum(sr_ref[...], axis=0, keepdims=True) * (1.0 / N_SEEDS)
    gw = jnp.dot(g, wg_ref[...], preferred_element_type=jnp.float32,
                 precision=lax.Precision.HIGHEST)
    s = e0_ref[0] + e1_ref[0]
    y = jnp.dot(s, we_ref[...], preferred_element_type=jnp.float32,
                precision=lax.Precision.HIGHEST)
    es = jnp.maximum(y + gw + be_ref[...], 0.0)
    es_ref[...] = es
    out_ref[...] = jnp.dot(es, wfc_ref[...], preferred_element_type=jnp.float32,
                           precision=lax.Precision.HIGHEST) + bfc_ref[...]

  return pl.pallas_call(
      body,
      grid=(n // _BLK,),
      in_specs=[
          pl.BlockSpec((1, _BLK, D), lambda i: (0, i, 0)),
          pl.BlockSpec((1, _BLK, D), lambda i: (1, i, 0)),
          pl.BlockSpec((N_SEEDS, D), lambda i: (0, 0)),
          pl.BlockSpec((D, D), lambda i: (0, 0)),
          pl.BlockSpec((1, D), lambda i: (0, 0)),
          pl.BlockSpec((D, D), lambda i: (0, 0)),
          pl.BlockSpec((D, D), lambda i: (0, 0)),
          pl.BlockSpec((1, D), lambda i: (0, 0)),
      ],
      out_specs=[
          pl.BlockSpec((_BLK, D), lambda i: (i, 0)),
          pl.BlockSpec((_BLK, D), lambda i: (i, 0)),
      ],
      out_shape=[
          jax.ShapeDtypeStruct((n, D), jnp.float32),
          jax.ShapeDtypeStruct((n, D), jnp.float32),
      ],
  )(parts, parts, seed_rows, W_e, b_e.reshape(1, D), W_g, W_fc_pad, b_fc_pad)


def kernel(seeds, es, ps, ep_adj, pe_adj, W_p1, b_p1, W_e1, b_e1, W_g1,
           W_p2, b_p2, W_e2, b_e2, W_g2, W_fc, b_fc):
  del ps
  def _pack(idx):
    return idx.reshape(NW, N_CHUNKS, CHUNK)

  pe_dst = _pack(pe_adj[0])
  pe_src = _pack(pe_adj[1])
  ep_dst = _pack(ep_adj[0])
  ep_src = _pack(ep_adj[1])
  zeros = jnp.zeros((N_P, D), jnp.float32)
  W_fc_pad = jnp.pad(W_fc, ((0, 0), (0, D - C)))
  b_fc_pad = jnp.pad(b_fc, (0, D - C)).reshape(1, D)

  seg_seed = _make_seg_sum(N_P, True)
  seg_plain = _make_seg_sum(N_E, False)

  p_parts, seed1 = seg_seed(es, pe_src, pe_dst, zeros, seeds)
  ps1 = _dense_pattern(p_parts, W_p1, b_p1)
  (e_parts,) = seg_plain(ps1, ep_src, ep_dst, zeros)
  es1 = _dense_entity(e_parts, seed1, W_e1, b_e1, W_g1)

  p_parts2, seed2 = seg_seed(es1, pe_src, pe_dst, zeros, seeds)
  ps2 = _dense_pattern(p_parts2, W_p2, b_p2)
  (e_parts2,) = seg_plain(ps2, ep_src, ep_dst, zeros)
  es2, out_pad = _dense_entity_fc(e_parts2, seed2, W_e2, b_e2, W_g2,
                                  W_fc_pad, b_fc_pad)
  return (out_pad[:, :C], es2, ps2)


# speedup vs baseline: 1.0189x; 1.0189x over previous
import functools

import jax
import jax.numpy as jnp
from jax import lax
from jax.experimental import pallas as pl
from jax.experimental.pallas import tpu as pltpu
from jax.experimental.pallas import tpu_sc as plsc

N_E = 10000
N_P = 10000
E = 320000
D = 128
C = 16
N_SEEDS = 160

NC = 2
NS = 16
NW = NC * NS
CHUNK = 125
N_CHUNKS = 80
EDGES_PER_TILE = E // NW


@functools.cache
def _make_seg_sum(n_seg, with_seeds):
  mesh = plsc.VectorSubcoreMesh(core_axis_name="c", subcore_axis_name="s")
  out_type = [jax.ShapeDtypeStruct((NC, n_seg, D), jnp.float32)]
  if with_seeds:
    out_type.append(jax.ShapeDtypeStruct((N_SEEDS, D), jnp.float32))
  rows_a = 632
  rows_b = n_seg - (NS - 1) * rows_a

  @functools.partial(
      pl.kernel,
      mesh=mesh,
      out_type=tuple(out_type),
      scratch_types=[
          pltpu.VMEM((N_CHUNKS, CHUNK), jnp.int32),
          pltpu.VMEM((N_CHUNKS, CHUNK), jnp.int32),
          pltpu.VMEM((CHUNK, D), jnp.float32),
          pltpu.VMEM_SHARED((n_seg, D), jnp.float32),
          pltpu.SemaphoreType.DMA,
      ],
  )
  def seg_sum(*refs):
    if with_seeds:
      (vals_hbm, sidx_hbm, didx_hbm, zeros_hbm, seeds_hbm,
       out_hbm, seed_out_hbm, S, I, rows, acc, sem) = refs
    else:
      (vals_hbm, sidx_hbm, didx_hbm, zeros_hbm,
       out_hbm, S, I, rows, acc, sem) = refs
    cid = lax.axis_index("c")
    sid = lax.axis_index("s")
    wid = cid * NS + sid
    r0 = pl.multiple_of(sid * rows_a, 8)

    pltpu.async_copy(sidx_hbm.at[wid, pl.ds(0, N_CHUNKS)], S, sem)
    pltpu.async_copy(didx_hbm.at[wid, pl.ds(0, N_CHUNKS)], I, sem)

    @pl.when(sid < NS - 1)
    def _zero_a():
      pltpu.sync_copy(zeros_hbm.at[pl.ds(r0, rows_a)],
                      acc.at[pl.ds(r0, rows_a)])

    @pl.when(sid == NS - 1)
    def _zero_b():
      pltpu.sync_copy(zeros_hbm.at[pl.ds(r0, rows_b)],
                      acc.at[pl.ds(r0, rows_b)])

    pltpu.make_async_copy(sidx_hbm.at[wid, pl.ds(0, N_CHUNKS)], S, sem).wait()
    pltpu.make_async_copy(didx_hbm.at[wid, pl.ds(0, N_CHUNKS)], I, sem).wait()
    plsc.subcore_barrier()

    def chunk_body(j, carry):
      pltpu.async_copy(vals_hbm.at[S.at[j]], rows, sem).wait()
      pltpu.sync_copy(rows, acc.at[I.at[j]], add=True)
      return carry

    lax.fori_loop(0, N_CHUNKS, chunk_body, 0)

    if with_seeds:
      @pl.when(wid < 2)
      def _gather_seeds():
        off = pl.multiple_of(wid * (N_SEEDS // 2), 8)
        pltpu.sync_copy(seeds_hbm.at[pl.ds(off, 80)], S.at[0, pl.ds(0, 80)])
        pltpu.async_copy(vals_hbm.at[S.at[0, pl.ds(0, 80)]],
                         rows.at[pl.ds(0, 80)], sem).wait()
        pltpu.sync_copy(rows.at[pl.ds(0, 80)],
                        seed_out_hbm.at[pl.ds(off, 80)])

    plsc.subcore_barrier()

    @pl.when(sid < NS - 1)
    def _write_a():
      pltpu.sync_copy(acc.at[pl.ds(r0, rows_a)],
                      out_hbm.at[cid, pl.ds(r0, rows_a)])

    @pl.when(sid == NS - 1)
    def _write_b():
      pltpu.sync_copy(acc.at[pl.ds(r0, rows_b)],
                      out_hbm.at[cid, pl.ds(r0, rows_b)])

  return seg_sum


_BLK = 1000


def _dense_pattern(parts, W, b):
  n = parts.shape[1]

  def body(p0_ref, p1_ref, w_ref, b_ref, o_ref):
    s = p0_ref[0] + p1_ref[0]
    y = jnp.dot(s, w_ref[...], preferred_element_type=jnp.float32,
                precision=lax.Precision.HIGHEST)
    o_ref[...] = jnp.maximum(y + b_ref[...], 0.0)

  return pl.pallas_call(
      body,
      grid=(n // _BLK,),
      in_specs=[
          pl.BlockSpec((1, _BLK, D), lambda i: (0, i, 0)),
          pl.BlockSpec((1, _BLK, D), lambda i: (1, i, 0)),
          pl.BlockSpec((D, D), lambda i: (0, 0)),
          pl.BlockSpec((1, D), lambda i: (0, 0)),
      ],
      out_specs=pl.BlockSpec((_BLK, D), lambda i: (i, 0)),
      out_shape=jax.ShapeDtypeStruct((n, D), jnp.float32),
  )(parts, parts, W, b.reshape(1, D))


def _dense_entity(parts, seed_rows, W_e, b_e, W_g):
  n = parts.shape[1]

  def body(e0_ref, e1_ref, sr_ref, we_ref, be_ref, wg_ref, o_ref):
    g = jnp.sum(sr_ref[...], axis=0, keepdims=True) * (1.0 / N_SEEDS)
    gw = jnp.dot(g, wg_ref[...], preferred_element_type=jnp.float32,
                 precision=lax.Precision.HIGHEST)
    s = e0_ref[0] + e1_ref[0]
    y = jnp.dot(s, we_ref[...], preferred_element_type=jnp.float32,
                precision=lax.Precision.HIGHEST)
    o_ref[...] = jnp.maximum(y + gw + be_ref[...], 0.0)

  return pl.pallas_call(
      body,
      grid=(n // _BLK,),
      in_specs=[
          pl.BlockSpec((1, _BLK, D), lambda i: (0, i, 0)),
          pl.BlockSpec((1, _BLK, D), lambda i: (1, i, 0)),
          pl.BlockSpec((N_SEEDS, D), lambda i: (0, 0)),
          pl.BlockSpec((D, D), lambda i: (0, 0)),
          pl.BlockSpec((1, D), lambda i: (0, 0)),
          pl.BlockSpec((D, D), lambda i: (0, 0)),
      ],
      out_specs=pl.BlockSpec((_BLK, D), lambda i: (i, 0)),
      out_shape=jax.ShapeDtypeStruct((n, D), jnp.float32),
  )(parts, parts, seed_rows, W_e, b_e.reshape(1, D), W_g)


def _dense_entity_fc(parts, seed_rows, W_e, b_e, W_g, W_fc_pad, b_fc_pad):
  n = parts.shape[1]

  def body(e0_ref, e1_ref, sr_ref, we_ref, be_ref, wg_ref, wfc_ref, bfc_ref,
           es_ref, out_ref):
    g = jnp.sum(sr_ref[...], axis=0, keepdims=True) * (1.0 / N_SEEDS)
    gw = jnp.dot(g, wg_ref[...], preferred_element_type=jnp.float32,
                 precision=lax.Precision.HIGHEST)
    s = e0_ref[0] + e1_ref[0]
    y = jnp.dot(s, we_ref[...], preferred_element_type=jnp.float32,
                precision=lax.Precision.HIGHEST)
    es = jnp.maximum(y + gw + be_ref[...], 0.0)
    es_ref[...] = es
    out_ref[...] = jnp.dot(es, wfc_ref[...], preferred_element_type=jnp.float32,
                           precision=lax.Precision.HIGHEST) + bfc_ref[...]

  return pl.pallas_call(
      body,
      grid=(n // _BLK,),
      in_specs=[
          pl.BlockSpec((1, _BLK, D), lambda i: (0, i, 0)),
          pl.BlockSpec((1, _BLK, D), lambda i: (1, i, 0)),
          pl.BlockSpec((N_SEEDS, D), lambda i: (0, 0)),
          pl.BlockSpec((D, D), lambda i: (0, 0)),
          pl.BlockSpec((1, D), lambda i: (0, 0)),
          pl.BlockSpec((D, D), lambda i: (0, 0)),
          pl.BlockSpec((D, D), lambda i: (0, 0)),
          pl.BlockSpec((1, D), lambda i: (0, 0)),
      ],
      out_specs=[
          pl.BlockSpec((_BLK, D), lambda i: (i, 0)),
          pl.BlockSpec((_BLK, D), lambda i: (i, 0)),
      ],
      out_shape=[
          jax.ShapeDtypeStruct((n, D), jnp.float32),
          jax.ShapeDtypeStruct((n, D), jnp.float32),
      ],
  )(parts, parts, seed_rows, W_e, b_e.reshape(1, D), W_g, W_fc_pad, b_fc_pad)


def kernel(seeds, es, ps, ep_adj, pe_adj, W_p1, b_p1, W_e1, b_e1, W_g1,
           W_p2, b_p2, W_e2, b_e2, W_g2, W_fc, b_fc):
  del ps
  def _pack(idx):
    return idx.reshape(NW, N_CHUNKS, CHUNK)

  pe_dst = _pack(pe_adj[0])
  pe_src = _pack(pe_adj[1])
  ep_dst = _pack(ep_adj[0])
  ep_src = _pack(ep_adj[1])
  zeros = jnp.zeros((N_P, D), jnp.float32)
  W_fc_pad = jnp.pad(W_fc, ((0, 0), (0, D - C)))
  b_fc_pad = jnp.pad(b_fc, (0, D - C)).reshape(1, D)

  seg_seed = _make_seg_sum(N_P, True)
  seg_plain = _make_seg_sum(N_E, False)

  p_parts, seed1 = seg_seed(es, pe_src, pe_dst, zeros, seeds)
  ps1 = _dense_pattern(p_parts, W_p1, b_p1)
  (e_parts,) = seg_plain(ps1, ep_src, ep_dst, zeros)
  es1 = _dense_entity(e_parts, seed1, W_e1, b_e1, W_g1)

  p_parts2, seed2 = seg_seed(es1, pe_src, pe_dst, zeros, seeds)
  ps2 = _dense_pattern(p_parts2, W_p2, b_p2)
  (e_parts2,) = seg_plain(ps2, ep_src, ep_dst, zeros)
  es2, out_pad = _dense_entity_fc(e_parts2, seed2, W_e2, b_e2, W_g2,
                                  W_fc_pad, b_fc_pad)
  return (out_pad[:, :C], es2, ps2)


# speedup vs baseline: 1.0601x; 1.0404x over previous
import functools

import jax
import jax.numpy as jnp
from jax import lax
from jax.experimental import pallas as pl
from jax.experimental.pallas import tpu as pltpu
from jax.experimental.pallas import tpu_sc as plsc

N_E = 10000
N_P = 10000
E = 320000
D = 128
C = 16
N_SEEDS = 160

NC = 2
NS = 16
NW = NC * NS
CHUNK = 125
N_CHUNKS = 80
EDGES_PER_TILE = E // NW


@functools.cache
def _make_seg_sum(n_seg, with_seeds):
  mesh = plsc.VectorSubcoreMesh(core_axis_name="c", subcore_axis_name="s")
  out_type = [jax.ShapeDtypeStruct((NC, n_seg, D), jnp.float32)]
  if with_seeds:
    out_type.append(jax.ShapeDtypeStruct((N_SEEDS, D), jnp.float32))
  rows_a = 632
  rows_b = n_seg - (NS - 1) * rows_a

  @functools.partial(
      pl.kernel,
      mesh=mesh,
      out_type=tuple(out_type),
      scratch_types=[
          pltpu.VMEM((N_CHUNKS, CHUNK), jnp.int32),
          pltpu.VMEM((N_CHUNKS, CHUNK), jnp.int32),
          pltpu.VMEM((CHUNK, D), jnp.float32),
          pltpu.VMEM_SHARED((n_seg, D), jnp.float32),
          pltpu.SemaphoreType.DMA,
      ],
  )
  def seg_sum(*refs):
    if with_seeds:
      (vals_hbm, sidx_hbm, didx_hbm, zeros_hbm, seeds_hbm,
       out_hbm, seed_out_hbm, S, I, rows, acc, sem) = refs
    else:
      (vals_hbm, sidx_hbm, didx_hbm, zeros_hbm,
       out_hbm, S, I, rows, acc, sem) = refs
    cid = lax.axis_index("c")
    sid = lax.axis_index("s")
    wid = cid * NS + sid
    r0 = pl.multiple_of(sid * rows_a, 8)

    pltpu.async_copy(sidx_hbm.at[wid, pl.ds(0, N_CHUNKS)], S, sem)
    pltpu.async_copy(didx_hbm.at[wid, pl.ds(0, N_CHUNKS)], I, sem)

    @pl.when(sid < NS - 1)
    def _zero_a():
      pltpu.sync_copy(zeros_hbm.at[pl.ds(r0, rows_a)],
                      acc.at[pl.ds(r0, rows_a)])

    @pl.when(sid == NS - 1)
    def _zero_b():
      pltpu.sync_copy(zeros_hbm.at[pl.ds(r0, rows_b)],
                      acc.at[pl.ds(r0, rows_b)])

    pltpu.make_async_copy(sidx_hbm.at[wid, pl.ds(0, N_CHUNKS)], S, sem).wait()
    pltpu.make_async_copy(didx_hbm.at[wid, pl.ds(0, N_CHUNKS)], I, sem).wait()
    plsc.subcore_barrier()

    def chunk_body(j, carry):
      pltpu.async_copy(vals_hbm.at[S.at[j]], rows, sem).wait()
      pltpu.sync_copy(rows, acc.at[I.at[j]], add=True)
      return carry

    lax.fori_loop(0, N_CHUNKS, chunk_body, 0)

    if with_seeds:
      @pl.when(wid < 2)
      def _gather_seeds():
        off = pl.multiple_of(wid * (N_SEEDS // 2), 8)
        pltpu.sync_copy(seeds_hbm.at[pl.ds(off, 80)], S.at[0, pl.ds(0, 80)])
        pltpu.async_copy(vals_hbm.at[S.at[0, pl.ds(0, 80)]],
                         rows.at[pl.ds(0, 80)], sem).wait()
        pltpu.sync_copy(rows.at[pl.ds(0, 80)],
                        seed_out_hbm.at[pl.ds(off, 80)])

    plsc.subcore_barrier()

    @pl.when(sid < NS - 1)
    def _write_a():
      pltpu.sync_copy(acc.at[pl.ds(r0, rows_a)],
                      out_hbm.at[cid, pl.ds(r0, rows_a)])

    @pl.when(sid == NS - 1)
    def _write_b():
      pltpu.sync_copy(acc.at[pl.ds(r0, rows_b)],
                      out_hbm.at[cid, pl.ds(r0, rows_b)])

  return seg_sum


_BLK = 1000


def _dense_pattern(parts, W, b):
  n = parts.shape[1]

  def body(p0_ref, p1_ref, w_ref, b_ref, o_ref):
    s = p0_ref[0] + p1_ref[0]
    y = jnp.dot(s, w_ref[...], preferred_element_type=jnp.float32,
                precision=lax.Precision.DEFAULT)
    o_ref[...] = jnp.maximum(y + b_ref[...], 0.0)

  return pl.pallas_call(
      body,
      grid=(n // _BLK,),
      in_specs=[
          pl.BlockSpec((1, _BLK, D), lambda i: (0, i, 0)),
          pl.BlockSpec((1, _BLK, D), lambda i: (1, i, 0)),
          pl.BlockSpec((D, D), lambda i: (0, 0)),
          pl.BlockSpec((1, D), lambda i: (0, 0)),
      ],
      out_specs=pl.BlockSpec((_BLK, D), lambda i: (i, 0)),
      out_shape=jax.ShapeDtypeStruct((n, D), jnp.float32),
  )(parts, parts, W, b.reshape(1, D))


def _dense_entity(parts, seed_rows, W_e, b_e, W_g):
  n = parts.shape[1]

  def body(e0_ref, e1_ref, sr_ref, we_ref, be_ref, wg_ref, o_ref):
    g = jnp.sum(sr_ref[...], axis=0, keepdims=True) * (1.0 / N_SEEDS)
    gw = jnp.dot(g, wg_ref[...], preferred_element_type=jnp.float32,
                 precision=lax.Precision.DEFAULT)
    s = e0_ref[0] + e1_ref[0]
    y = jnp.dot(s, we_ref[...], preferred_element_type=jnp.float32,
                precision=lax.Precision.DEFAULT)
    o_ref[...] = jnp.maximum(y + gw + be_ref[...], 0.0)

  return pl.pallas_call(
      body,
      grid=(n // _BLK,),
      in_specs=[
          pl.BlockSpec((1, _BLK, D), lambda i: (0, i, 0)),
          pl.BlockSpec((1, _BLK, D), lambda i: (1, i, 0)),
          pl.BlockSpec((N_SEEDS, D), lambda i: (0, 0)),
          pl.BlockSpec((D, D), lambda i: (0, 0)),
          pl.BlockSpec((1, D), lambda i: (0, 0)),
          pl.BlockSpec((D, D), lambda i: (0, 0)),
      ],
      out_specs=pl.BlockSpec((_BLK, D), lambda i: (i, 0)),
      out_shape=jax.ShapeDtypeStruct((n, D), jnp.float32),
  )(parts, parts, seed_rows, W_e, b_e.reshape(1, D), W_g)


def _dense_entity_fc(parts, seed_rows, W_e, b_e, W_g, W_fc, b_fc):
  n = parts.shape[1]

  def body(e0_ref, e1_ref, sr_ref, we_ref, be_ref, wg_ref, wfc_ref, bfc_ref,
           es_ref, out_ref):
    g = jnp.sum(sr_ref[...], axis=0, keepdims=True) * (1.0 / N_SEEDS)
    gw = jnp.dot(g, wg_ref[...], preferred_element_type=jnp.float32,
                 precision=lax.Precision.DEFAULT)
    s = e0_ref[0] + e1_ref[0]
    y = jnp.dot(s, we_ref[...], preferred_element_type=jnp.float32,
                precision=lax.Precision.DEFAULT)
    es = jnp.maximum(y + gw + be_ref[...], 0.0)
    es_ref[...] = es
    out_ref[...] = jnp.dot(es, wfc_ref[...], preferred_element_type=jnp.float32,
                           precision=lax.Precision.DEFAULT) + bfc_ref[...]

  return pl.pallas_call(
      body,
      grid=(n // _BLK,),
      in_specs=[
          pl.BlockSpec((1, _BLK, D), lambda i: (0, i, 0)),
          pl.BlockSpec((1, _BLK, D), lambda i: (1, i, 0)),
          pl.BlockSpec((N_SEEDS, D), lambda i: (0, 0)),
          pl.BlockSpec((D, D), lambda i: (0, 0)),
          pl.BlockSpec((1, D), lambda i: (0, 0)),
          pl.BlockSpec((D, D), lambda i: (0, 0)),
          pl.BlockSpec((D, C), lambda i: (0, 0)),
          pl.BlockSpec((1, C), lambda i: (0, 0)),
      ],
      out_specs=[
          pl.BlockSpec((_BLK, D), lambda i: (i, 0)),
          pl.BlockSpec((_BLK, C), lambda i: (i, 0)),
      ],
      out_shape=[
          jax.ShapeDtypeStruct((n, D), jnp.float32),
          jax.ShapeDtypeStruct((n, C), jnp.float32),
      ],
  )(parts, parts, seed_rows, W_e, b_e.reshape(1, D), W_g, W_fc, b_fc.reshape(1, C))


def kernel(seeds, es, ps, ep_adj, pe_adj, W_p1, b_p1, W_e1, b_e1, W_g1,
           W_p2, b_p2, W_e2, b_e2, W_g2, W_fc, b_fc):
  del ps
  def _pack(idx):
    return idx.reshape(NW, N_CHUNKS, CHUNK)

  pe_dst = _pack(pe_adj[0])
  pe_src = _pack(pe_adj[1])
  ep_dst = _pack(ep_adj[0])
  ep_src = _pack(ep_adj[1])
  zeros = jnp.zeros((N_P, D), jnp.float32)

  seg_seed = _make_seg_sum(N_P, True)
  seg_plain = _make_seg_sum(N_E, False)

  p_parts, seed1 = seg_seed(es, pe_src, pe_dst, zeros, seeds)
  ps1 = _dense_pattern(p_parts, W_p1, b_p1)
  (e_parts,) = seg_plain(ps1, ep_src, ep_dst, zeros)
  es1 = _dense_entity(e_parts, seed1, W_e1, b_e1, W_g1)

  p_parts2, seed2 = seg_seed(es1, pe_src, pe_dst, zeros, seeds)
  ps2 = _dense_pattern(p_parts2, W_p2, b_p2)
  (e_parts2,) = seg_plain(ps2, ep_src, ep_dst, zeros)
  es2, out = _dense_entity_fc(e_parts2, seed2, W_e2, b_e2, W_g2, W_fc, b_fc)
  return (out, es2, ps2)
